# baseline (speedup 1.0000x reference)
import functools
import jax
import jax.numpy as jnp
from jax import lax
from jax.experimental import pallas as pl
from jax.experimental.pallas import tpu as pltpu
from jax.experimental.pallas import tpu_sc as plsc

N = 10000
NP = 10240
E = 160000
D = 128
NK = 16
NTILES = 16
EPT = E // NTILES
BB = 80
NB = EPT // BB
ROWS_PT = NP // NTILES
NROUND = NK // 2
NPAIR = (NB - 1) // 2

_sc_mesh = plsc.VectorSubcoreMesh(core_axis_name="c", subcore_axis_name="s")


def _sc_body(x_hbm, h_hbm, reci_hbm, recf_hbm,
             gx_hbm, gh_hbm, deg_hbm,
             idx0, idx1, idx2, att0, att1, att2, xs0, xs1, xs2,
             sidx0, sidx1, sidx2, zbuf, acc,
             rsem0, rsem1, rsem2, gsem0, gsem1, gsem2,
             ssem0, ssem1, ssem2):
    c = lax.axis_index("c")
    s = lax.axis_index("s")

    idxb = (idx0, idx1, idx2)
    attb = (att0, att1, att2)
    xsb = (xs0, xs1, xs2)
    sidxb = (sidx0, sidx1, sidx2)
    rsem = (rsem0, rsem1, rsem2)
    gsem = (gsem0, gsem1, gsem2)
    ssem = (ssem0, ssem1, ssem2)

    def _zb(i, carry):
        for j in range(8):
            zbuf[i, pl.ds(j * 16, 16)] = jnp.zeros((16,), jnp.float32)
        return carry
    lax.fori_loop(0, 16, _zb, 0)

    for k in range(40):
        pltpu.sync_copy(zbuf, acc.at[pl.ds(s * ROWS_PT + k * 16, 16)])

    plsc.subcore_barrier()

    def rec_sync(b, p):
        pltpu.sync_copy(reci_hbm.at[s].at[b], idxb[p])
        pltpu.sync_copy(recf_hbm.at[s].at[b], attb[p])

    def rec_issue(b, p):
        pltpu.async_copy(reci_hbm.at[s].at[b], idxb[p], rsem[p])
        pltpu.async_copy(recf_hbm.at[s].at[b], attb[p], rsem[p])

    def rec_wait(p):
        pltpu.make_async_copy(reci_hbm.at[s].at[0], idxb[p], rsem[p]).wait()
        pltpu.make_async_copy(recf_hbm.at[s].at[0], attb[p], rsem[p]).wait()

    def gather_issue(feat_hbm, p):
        pltpu.async_copy(feat_hbm.at[idxb[p].at[0]], xsb[p], gsem[p])

    def gather_wait(feat_hbm, p):
        pltpu.make_async_copy(feat_hbm.at[idxb[p].at[0]], xsb[p],
                              gsem[p]).wait()

    def scatter_issue(p):
        pltpu.async_copy(xsb[p], acc.at[sidxb[p]], ssem[p], add=True)

    def scatter_wait(p):
        pltpu.make_async_copy(xsb[p], acc.at[sidxb[p]], ssem[p]).wait()

    def copy_sidx(p):
        for k5 in range(BB // 16):
            sidxb[p][pl.ds(k5 * 16, 16)] = idxb[p][1, pl.ds(k5 * 16, 16)]

    def do_feature(feat_hbm, g_hbm):
        def round_body(r, carry):
            m = 2 * r + c
            t0 = lax.convert_element_type(c, jnp.float32)
            t1 = lax.convert_element_type(r & 1, jnp.float32)
            t2 = lax.convert_element_type((r >> 1) & 1, jnp.float32)
            t3 = lax.convert_element_type((r >> 2) & 1, jnp.float32)

            def compute(p):
                for g in range(BB // 16):
                    off = g * 16
                    u0 = jnp.clip(attb[p][0, pl.ds(off, 16)], 0.0, 1.0)
                    u1 = jnp.clip(attb[p][1, pl.ds(off, 16)], 0.0, 1.0)
                    u2 = jnp.clip(attb[p][2, pl.ds(off, 16)], 0.0, 1.0)
                    u3 = jnp.clip(attb[p][3, pl.ds(off, 16)], 0.0, 1.0)
                    f0 = (1.0 - u0) + t0 * (u0 + u0 - 1.0)
                    f1 = (1.0 - u1) + t1 * (u1 + u1 - 1.0)
                    f2 = (1.0 - u2) + t2 * (u2 + u2 - 1.0)
                    f3 = (1.0 - u3) + t3 * (u3 + u3 - 1.0)
                    basis = (f0 * f1) * (f2 * f3)
                    for e in range(16):
                        erow = g * 16 + e
                        sb = basis[e]
                        for j in range(8):
                            xsb[p][erow, pl.ds(j * 16, 16)] = (
                                xsb[p][erow, pl.ds(j * 16, 16)] * sb)

            rec_sync(0, 0)
            rec_sync(1, 1)
            rec_sync(2, 2)
            gather_issue(feat_hbm, 0)
            gather_issue(feat_hbm, 1)

            def triple(k, carry2):
                for i in range(3):
                    b = 3 * k + i
                    p = i % 3
                    pn = (p + 1) % 3
                    pg = (p + 2) % 3

                    @pl.when(b <= NB - 1)
                    def _sub():
                        gather_wait(feat_hbm, p)

                        @pl.when(b >= 2)
                        def _sw():
                            scatter_wait(pn)

                        @pl.when(b + 2 <= NB - 1)
                        def _gi():
                            @pl.when(b >= 1)
                            def _rw():
                                rec_wait(pg)
                            gather_issue(feat_hbm, pg)
                        compute(p)
                        copy_sidx(p)
                        scatter_issue(p)

                        @pl.when(b + 3 <= NB - 1)
                        def _ri():
                            rec_issue(b + 3, p)
                return carry2

            lax.fori_loop(0, (NB + 3) // 3, triple, 0)
            scatter_wait(0)
            scatter_wait(1)

            plsc.subcore_barrier()
            pltpu.sync_copy(acc.at[pl.ds(s * ROWS_PT, ROWS_PT)],
                            g_hbm.at[m, pl.ds(s * ROWS_PT, ROWS_PT)])
            for k in range(40):
                pltpu.sync_copy(zbuf,
                                acc.at[pl.ds(s * ROWS_PT + k * 16, 16)])
            plsc.subcore_barrier()
            return carry

        lax.fori_loop(0, NROUND, round_body, 0)

    do_feature(x_hbm, gx_hbm)

    @pl.when(c == 0)
    def _deg_scatter():
        def _ones(i, carry):
            for j in range(8):
                xs0[i, pl.ds(j * 16, 16)] = jnp.ones((16,), jnp.float32)
            return carry
        lax.fori_loop(0, BB, _ones, 0)

        def dscatter_issue(p):
            pltpu.async_copy(xs0, acc.at[sidxb[p]], ssem[p], add=True)

        def dscatter_wait(p):
            pltpu.make_async_copy(xs0, acc.at[sidxb[p]], ssem[p]).wait()

        rec_sync(0, 0)
        rec_issue(1, 1)

        def dpair(k, carry):
            b0 = 2 * k

            @pl.when(k > 0)
            def _w0():
                dscatter_wait(0)
            copy_sidx(0)
            dscatter_issue(0)
            rec_wait(1)
            rec_issue(b0 + 2, 0)

            @pl.when(k > 0)
            def _w1():
                dscatter_wait(1)
            copy_sidx(1)
            dscatter_issue(1)
            rec_wait(0)

            @pl.when(k < NPAIR - 1)
            def _ri():
                rec_issue(b0 + 3, 1)
            return carry

        lax.fori_loop(0, NPAIR, dpair, 0)
        dscatter_wait(0)
        copy_sidx(0)
        dscatter_issue(0)
        dscatter_wait(1)
        dscatter_wait(0)

    plsc.subcore_barrier()

    @pl.when(c == 0)
    def _deg_wout():
        pltpu.sync_copy(acc.at[pl.ds(s * ROWS_PT, ROWS_PT)],
                        deg_hbm.at[pl.ds(s * ROWS_PT, ROWS_PT)])
        for k in range(40):
            pltpu.sync_copy(zbuf, acc.at[pl.ds(s * ROWS_PT + k * 16, 16)])
    plsc.subcore_barrier()

    do_feature(h_hbm, gh_hbm)


_sc_build = functools.partial(
    pl.kernel,
    out_type=(jax.ShapeDtypeStruct((NK, NP, D), jnp.float32),
              jax.ShapeDtypeStruct((NK, NP, D), jnp.float32),
              jax.ShapeDtypeStruct((NP, D), jnp.float32)),
    mesh=_sc_mesh,
    scratch_types=[
        pltpu.VMEM((2, BB), jnp.int32),
        pltpu.VMEM((2, BB), jnp.int32),
        pltpu.VMEM((2, BB), jnp.int32),
        pltpu.VMEM((4, BB), jnp.float32),
        pltpu.VMEM((4, BB), jnp.float32),
        pltpu.VMEM((4, BB), jnp.float32),
        pltpu.VMEM((BB, D), jnp.float32),
        pltpu.VMEM((BB, D), jnp.float32),
        pltpu.VMEM((BB, D), jnp.float32),
        pltpu.VMEM((BB,), jnp.int32),
        pltpu.VMEM((BB,), jnp.int32),
        pltpu.VMEM((BB,), jnp.int32),
        pltpu.VMEM((16, D), jnp.float32),
        pltpu.VMEM_SHARED((NP, D), jnp.float32),
        pltpu.SemaphoreType.DMA,
        pltpu.SemaphoreType.DMA,
        pltpu.SemaphoreType.DMA,
        pltpu.SemaphoreType.DMA,
        pltpu.SemaphoreType.DMA,
        pltpu.SemaphoreType.DMA,
        pltpu.SemaphoreType.DMA,
        pltpu.SemaphoreType.DMA,
        pltpu.SemaphoreType.DMA,
    ],
    name="sc_spline_segsum",
)(_sc_body)


BLK = 400
NBLK = N // BLK


def _tc_body(gx_ref, gh_ref, x_ref, h_ref, deg_ref, wx_ref, wh_ref,
             rx_ref, rh_ref, bx_ref, bh_ref, out_ref):
    accx = jnp.zeros((BLK, 384), jnp.float32)
    acch = jnp.zeros((BLK, 256), jnp.float32)
    for m in range(NK):
        accx = accx + jnp.dot(gx_ref[m], wx_ref[m],
                              preferred_element_type=jnp.float32)
        acch = acch + jnp.dot(gh_ref[m], wh_ref[m],
                              preferred_element_type=jnp.float32)
    deg = deg_ref[:, 0:1]
    inv = 1.0 / jnp.clip(deg, 1.0, None)
    accx = accx * inv
    acch = acch * inv
    accx = accx + jnp.dot(x_ref[...], rx_ref[...],
                          preferred_element_type=jnp.float32) + bx_ref[...]
    acch = acch + jnp.dot(h_ref[...], rh_ref[...],
                          preferred_element_type=jnp.float32) + bh_ref[...]
    xr = accx[:, 0:128]
    xz = accx[:, 128:256]
    xn = accx[:, 256:384]
    hr = acch[:, 0:128]
    hz = acch[:, 128:256]
    rg = jax.nn.sigmoid(xr + hr)
    zg = jax.nn.sigmoid(xz + hz)
    ng = jnp.tanh(xn + rg * hr)
    out_ref[...] = (1.0 - zg) * ng + zg * h_ref[...]


def _tc_dense(gx, gh, x, hidden, deg16, wx, wh, rx, rh, bx, bh):
    return pl.pallas_call(
        _tc_body,
        grid=(NBLK,),
        in_specs=[
            pl.BlockSpec((NK, BLK, D), lambda i: (0, i, 0)),
            pl.BlockSpec((NK, BLK, D), lambda i: (0, i, 0)),
            pl.BlockSpec((BLK, D), lambda i: (i, 0)),
            pl.BlockSpec((BLK, D), lambda i: (i, 0)),
            pl.BlockSpec((BLK, D), lambda i: (i, 0)),
            pl.BlockSpec((NK, D, 384), lambda i: (0, 0, 0)),
            pl.BlockSpec((NK, D, 256), lambda i: (0, 0, 0)),
            pl.BlockSpec((D, 384), lambda i: (0, 0)),
            pl.BlockSpec((D, 256), lambda i: (0, 0)),
            pl.BlockSpec((1, 384), lambda i: (0, 0)),
            pl.BlockSpec((1, 256), lambda i: (0, 0)),
        ],
        out_specs=pl.BlockSpec((BLK, D), lambda i: (i, 0)),
        out_shape=jax.ShapeDtypeStruct((N, D), jnp.float32),
    )(gx, gh, x, hidden, deg16, wx, wh, rx, rh, bx, bh)


def kernel(x, hidden, edge_index, edge_attr,
           W_xr, Wroot_xr, b_xr, W_hr, Wroot_hr, b_hr,
           W_xz, Wroot_xz, b_xz, W_hz, Wroot_hz, b_hz,
           W_xn, Wroot_xn, b_xn):
    reci = edge_index.reshape(2, NTILES, NB, BB).transpose(1, 2, 0, 3)
    recf = edge_attr.reshape(NTILES, NB, BB, 4).transpose(0, 1, 3, 2)

    gx, gh, deg16 = _sc_build(x, hidden, reci, recf)

    wx = jnp.concatenate([W_xr, W_xz, W_xn], axis=2)
    wh = jnp.concatenate([W_hr, W_hz], axis=2)
    rx = jnp.concatenate([Wroot_xr, Wroot_xz, Wroot_xn], axis=1)
    rh = jnp.concatenate([Wroot_hr, Wroot_hz], axis=1)
    bx = jnp.concatenate([b_xr, b_xz, b_xn]).reshape(1, 384)
    bh = jnp.concatenate([b_hr, b_hz]).reshape(1, 256)

    return _tc_dense(gx, gh, x, hidden, deg16, wx, wh, rx, rh, bx, bh)


# speedup vs baseline: 1.4477x; 1.4477x over previous
import functools
import jax
import jax.numpy as jnp
from jax import lax
from jax.experimental import pallas as pl
from jax.experimental.pallas import tpu as pltpu
from jax.experimental.pallas import tpu_sc as plsc

N = 10000
NP = 10240
E = 160000
D = 128
NK = 16
NTILES = 16
EPT = E // NTILES
BB = 80
NB = EPT // BB
ROWS_PT = NP // NTILES
NROUND = NK // 2
NPAIR = (NB - 1) // 2

_sc_mesh = plsc.VectorSubcoreMesh(core_axis_name="c", subcore_axis_name="s")


def _sc_body(x_hbm, h_hbm, reci_hbm, recf_hbm,
             gx_hbm, gh_hbm, deg_hbm,
             idx0, idx1, att0, att1, xs0, xs1, sidx0, sidx1, zbuf,
             acc, rsem0, rsem1, gsem0, gsem1, ssem0, ssem1):
    c = lax.axis_index("c")
    s = lax.axis_index("s")

    idxb = (idx0, idx1)
    attb = (att0, att1)
    xsb = (xs0, xs1)
    sidxb = (sidx0, sidx1)
    rsem = (rsem0, rsem1)
    gsem = (gsem0, gsem1)
    ssem = (ssem0, ssem1)

    def _zb(i, carry):
        for j in range(8):
            zbuf[i, pl.ds(j * 16, 16)] = jnp.zeros((16,), jnp.float32)
        return carry
    lax.fori_loop(0, 32, _zb, 0)

    for k in range(20):
        pltpu.sync_copy(zbuf, acc.at[pl.ds(s * ROWS_PT + k * 32, 32)])

    plsc.subcore_barrier()

    def rec_sync(b, p):
        pltpu.sync_copy(reci_hbm.at[s].at[b], idxb[p])
        pltpu.sync_copy(recf_hbm.at[s].at[b], attb[p])

    def rec_issue(b, p):
        pltpu.async_copy(reci_hbm.at[s].at[b], idxb[p], rsem[p])
        pltpu.async_copy(recf_hbm.at[s].at[b], attb[p], rsem[p])

    def rec_wait(p):
        pltpu.make_async_copy(reci_hbm.at[s].at[0], idxb[p], rsem[p]).wait()
        pltpu.make_async_copy(recf_hbm.at[s].at[0], attb[p], rsem[p]).wait()

    def gather_issue(feat_hbm, p):
        pltpu.async_copy(feat_hbm.at[idxb[p].at[0]], xsb[p], gsem[p])

    def gather_wait(feat_hbm, p):
        pltpu.make_async_copy(feat_hbm.at[idxb[p].at[0]], xsb[p],
                              gsem[p]).wait()

    def scatter_issue(p):
        pltpu.async_copy(xsb[p], acc.at[sidxb[p]], ssem[p], add=True)

    def scatter_wait(p):
        pltpu.make_async_copy(xsb[p], acc.at[sidxb[p]], ssem[p]).wait()

    def copy_sidx(p):
        for k5 in range(BB // 16):
            sidxb[p][pl.ds(k5 * 16, 16)] = idxb[p][1, pl.ds(k5 * 16, 16)]

    def do_feature(feat_hbm, g_hbm):
        def round_body(r, carry):
            m = 2 * r + c
            t0 = lax.convert_element_type(c, jnp.float32)
            t1 = lax.convert_element_type(r & 1, jnp.float32)
            t2 = lax.convert_element_type((r >> 1) & 1, jnp.float32)
            t3 = lax.convert_element_type((r >> 2) & 1, jnp.float32)

            def compute(p):
                for g in range(BB // 16):
                    off = g * 16
                    u0 = jnp.clip(attb[p][0, pl.ds(off, 16)], 0.0, 1.0)
                    u1 = jnp.clip(attb[p][1, pl.ds(off, 16)], 0.0, 1.0)
                    u2 = jnp.clip(attb[p][2, pl.ds(off, 16)], 0.0, 1.0)
                    u3 = jnp.clip(attb[p][3, pl.ds(off, 16)], 0.0, 1.0)
                    f0 = (1.0 - u0) + t0 * (u0 + u0 - 1.0)
                    f1 = (1.0 - u1) + t1 * (u1 + u1 - 1.0)
                    f2 = (1.0 - u2) + t2 * (u2 + u2 - 1.0)
                    f3 = (1.0 - u3) + t3 * (u3 + u3 - 1.0)
                    basis = (f0 * f1) * (f2 * f3)
                    for e in range(16):
                        erow = g * 16 + e
                        sb = basis[e]
                        for j in range(8):
                            xsb[p][erow, pl.ds(j * 16, 16)] = (
                                xsb[p][erow, pl.ds(j * 16, 16)] * sb)

            rec_sync(0, 0)
            gather_issue(feat_hbm, 0)
            rec_issue(1, 1)

            def pair(k, carry2):
                b0 = 2 * k
                gather_wait(feat_hbm, 0)

                @pl.when(k > 0)
                def _w0():
                    scatter_wait(1)
                rec_wait(1)
                gather_issue(feat_hbm, 1)
                compute(0)
                copy_sidx(0)
                scatter_issue(0)
                rec_issue(b0 + 2, 0)
                gather_wait(feat_hbm, 1)
                scatter_wait(0)
                rec_wait(0)
                gather_issue(feat_hbm, 0)
                compute(1)
                copy_sidx(1)
                scatter_issue(1)

                @pl.when(k < NPAIR - 1)
                def _ri():
                    rec_issue(b0 + 3, 1)
                return carry2

            lax.fori_loop(0, NPAIR, pair, 0)
            gather_wait(feat_hbm, 0)
            compute(0)
            copy_sidx(0)
            scatter_issue(0)
            scatter_wait(1)
            scatter_wait(0)

            plsc.subcore_barrier()
            pltpu.sync_copy(acc.at[pl.ds(s * ROWS_PT, ROWS_PT)],
                            g_hbm.at[m, pl.ds(s * ROWS_PT, ROWS_PT)])
            for k in range(20):
                pltpu.sync_copy(zbuf,
                                acc.at[pl.ds(s * ROWS_PT + k * 32, 32)])
            plsc.subcore_barrier()
            return carry

        lax.fori_loop(0, NROUND, round_body, 0)

    do_feature(x_hbm, gx_hbm)

    @pl.when(c == 0)
    def _deg_scatter():
        def _ones(i, carry):
            for j in range(8):
                xs0[i, pl.ds(j * 16, 16)] = jnp.ones((16,), jnp.float32)
            return carry
        lax.fori_loop(0, BB, _ones, 0)

        def dscatter_issue(p):
            pltpu.async_copy(xs0, acc.at[sidxb[p]], ssem[p], add=True)

        def dscatter_wait(p):
            pltpu.make_async_copy(xs0, acc.at[sidxb[p]], ssem[p]).wait()

        rec_sync(0, 0)
        rec_issue(1, 1)

        def dpair(k, carry):
            b0 = 2 * k

            @pl.when(k > 0)
            def _w0():
                dscatter_wait(0)
            copy_sidx(0)
            dscatter_issue(0)
            rec_wait(1)
            rec_issue(b0 + 2, 0)

            @pl.when(k > 0)
            def _w1():
                dscatter_wait(1)
            copy_sidx(1)
            dscatter_issue(1)
            rec_wait(0)

            @pl.when(k < NPAIR - 1)
            def _ri():
                rec_issue(b0 + 3, 1)
            return carry

        lax.fori_loop(0, NPAIR, dpair, 0)
        dscatter_wait(0)
        copy_sidx(0)
        dscatter_issue(0)
        dscatter_wait(1)
        dscatter_wait(0)

    plsc.subcore_barrier()

    @pl.when(c == 0)
    def _deg_wout():
        pltpu.sync_copy(acc.at[pl.ds(s * ROWS_PT, ROWS_PT)],
                        deg_hbm.at[pl.ds(s * ROWS_PT, ROWS_PT)])
        for k in range(20):
            pltpu.sync_copy(zbuf, acc.at[pl.ds(s * ROWS_PT + k * 32, 32)])
    plsc.subcore_barrier()

    do_feature(h_hbm, gh_hbm)


_sc_build = functools.partial(
    pl.kernel,
    out_type=(jax.ShapeDtypeStruct((NK, NP, D), jnp.float32),
              jax.ShapeDtypeStruct((NK, NP, D), jnp.float32),
              jax.ShapeDtypeStruct((NP, D), jnp.float32)),
    mesh=_sc_mesh,
    scratch_types=[
        pltpu.VMEM((2, BB), jnp.int32),
        pltpu.VMEM((2, BB), jnp.int32),
        pltpu.VMEM((4, BB), jnp.float32),
        pltpu.VMEM((4, BB), jnp.float32),
        pltpu.VMEM((BB, D), jnp.float32),
        pltpu.VMEM((BB, D), jnp.float32),
        pltpu.VMEM((BB,), jnp.int32),
        pltpu.VMEM((BB,), jnp.int32),
        pltpu.VMEM((32, D), jnp.float32),
        pltpu.VMEM_SHARED((NP, D), jnp.float32),
        pltpu.SemaphoreType.DMA,
        pltpu.SemaphoreType.DMA,
        pltpu.SemaphoreType.DMA,
        pltpu.SemaphoreType.DMA,
        pltpu.SemaphoreType.DMA,
        pltpu.SemaphoreType.DMA,
    ],
    name="sc_spline_segsum",
)(_sc_body)


BLK = 400
NBLK = N // BLK


def _tc_body(gx_ref, gh_ref, x_ref, h_ref, deg_ref, wx_ref, wh_ref,
             rx_ref, rh_ref, bx_ref, bh_ref, out_ref):
    accx = jnp.zeros((BLK, 384), jnp.float32)
    acch = jnp.zeros((BLK, 256), jnp.float32)
    for m in range(NK):
        accx = accx + jnp.dot(gx_ref[m], wx_ref[m],
                              preferred_element_type=jnp.float32)
        acch = acch + jnp.dot(gh_ref[m], wh_ref[m],
                              preferred_element_type=jnp.float32)
    deg = deg_ref[:, 0:1]
    inv = 1.0 / jnp.clip(deg, 1.0, None)
    accx = accx * inv
    acch = acch * inv
    accx = accx + jnp.dot(x_ref[...], rx_ref[...],
                          preferred_element_type=jnp.float32) + bx_ref[...]
    acch = acch + jnp.dot(h_ref[...], rh_ref[...],
                          preferred_element_type=jnp.float32) + bh_ref[...]
    xr = accx[:, 0:128]
    xz = accx[:, 128:256]
    xn = accx[:, 256:384]
    hr = acch[:, 0:128]
    hz = acch[:, 128:256]
    rg = jax.nn.sigmoid(xr + hr)
    zg = jax.nn.sigmoid(xz + hz)
    ng = jnp.tanh(xn + rg * hr)
    out_ref[...] = (1.0 - zg) * ng + zg * h_ref[...]


def _tc_dense(gx, gh, x, hidden, deg16, wx, wh, rx, rh, bx, bh):
    return pl.pallas_call(
        _tc_body,
        grid=(NBLK,),
        in_specs=[
            pl.BlockSpec((NK, BLK, D), lambda i: (0, i, 0)),
            pl.BlockSpec((NK, BLK, D), lambda i: (0, i, 0)),
            pl.BlockSpec((BLK, D), lambda i: (i, 0)),
            pl.BlockSpec((BLK, D), lambda i: (i, 0)),
            pl.BlockSpec((BLK, D), lambda i: (i, 0)),
            pl.BlockSpec((NK, D, 384), lambda i: (0, 0, 0)),
            pl.BlockSpec((NK, D, 256), lambda i: (0, 0, 0)),
            pl.BlockSpec((D, 384), lambda i: (0, 0)),
            pl.BlockSpec((D, 256), lambda i: (0, 0)),
            pl.BlockSpec((1, 384), lambda i: (0, 0)),
            pl.BlockSpec((1, 256), lambda i: (0, 0)),
        ],
        out_specs=pl.BlockSpec((BLK, D), lambda i: (i, 0)),
        out_shape=jax.ShapeDtypeStruct((N, D), jnp.float32),
    )(gx, gh, x, hidden, deg16, wx, wh, rx, rh, bx, bh)


def kernel(x, hidden, edge_index, edge_attr,
           W_xr, Wroot_xr, b_xr, W_hr, Wroot_hr, b_hr,
           W_xz, Wroot_xz, b_xz, W_hz, Wroot_hz, b_hz,
           W_xn, Wroot_xn, b_xn):
    reci = edge_index.reshape(2, NTILES, NB, BB).transpose(1, 2, 0, 3)
    recf = edge_attr.reshape(NTILES, NB, BB, 4).transpose(0, 1, 3, 2)

    gx, gh, deg16 = _sc_build(x, hidden, reci, recf)

    wx = jnp.concatenate([W_xr, W_xz, W_xn], axis=2)
    wh = jnp.concatenate([W_hr, W_hz], axis=2)
    rx = jnp.concatenate([Wroot_xr, Wroot_xz, Wroot_xn], axis=1)
    rh = jnp.concatenate([Wroot_hr, Wroot_hz], axis=1)
    bx = jnp.concatenate([b_xr, b_xz, b_xn]).reshape(1, 384)
    bh = jnp.concatenate([b_hr, b_hz]).reshape(1, 256)

    return _tc_dense(gx, gh, x, hidden, deg16, wx, wh, rx, rh, bx, bh)
